# baseline (speedup 1.0000x reference)
import jax
import jax.numpy as jnp
from jax.experimental import pallas as pl

_K = 15
_INV_TEMP = 10.0
_EPS = 1e-12


def _pool_norm_kernel(a_ref, m_ref, an_ref, mn_ref):
    for src, dst in ((a_ref, an_ref), (m_ref, mn_ref)):
        x = jnp.mean(src[...], axis=1)
        nrm = jnp.sqrt(jnp.sum(x * x, axis=1, keepdims=True))
        dst[...] = x / jnp.maximum(nrm, _EPS)


def _loss_kernel(ar_ref, mr_ref, afull_ref, mfull_ref, out_ref):
    i = pl.program_id(0)
    rb, b = ar_ref.shape[0], afull_ref.shape[0]
    dn = (((1,), (1,)), ((), ()))
    sim_a = jax.lax.dot_general(ar_ref[...], afull_ref[...], dn,
                                preferred_element_type=jnp.float32)
    row_ids = i * rb + jax.lax.broadcasted_iota(jnp.int32, (rb, b), 0)
    col_ids = jax.lax.broadcasted_iota(jnp.int32, (rb, b), 1)
    is_diag = row_ids == col_ids
    neg_inf = jnp.float32(-jnp.inf)
    sim_a = jnp.where(is_diag, neg_inf, sim_a)

    work = sim_a
    for _ in range(_K - 1):
        mx = jnp.max(work, axis=1, keepdims=True)
        work = jnp.where(work >= mx, neg_inf, work)
    thr = jnp.max(work, axis=1, keepdims=True)
    pos = sim_a >= thr

    sim_m = jax.lax.dot_general(mr_ref[...], mfull_ref[...], dn,
                                preferred_element_type=jnp.float32) * _INV_TEMP
    smax = jnp.max(sim_m, axis=1, keepdims=True)
    e = jnp.exp(sim_m - smax)
    e = jnp.where(is_diag, 0.0, e)
    pos_sum = jnp.sum(jnp.where(pos, e, 0.0), axis=1) + 1e-8
    all_sum = jnp.sum(e, axis=1) + 1e-8
    contrib = jnp.sum(jnp.log(pos_sum) - jnp.log(all_sum))

    @pl.when(i == 0)
    def _():
        out_ref[...] = jnp.zeros_like(out_ref)

    out_ref[...] += jnp.reshape(-contrib / b, (1, 1))


def kernel(anchor, modality):
    B, S, D = anchor.shape
    BR = 128
    an, mn = pl.pallas_call(
        _pool_norm_kernel,
        grid=(B // BR,),
        in_specs=[
            pl.BlockSpec((BR, S, D), lambda i: (i, 0, 0)),
            pl.BlockSpec((BR, S, D), lambda i: (i, 0, 0)),
        ],
        out_specs=[
            pl.BlockSpec((BR, D), lambda i: (i, 0)),
            pl.BlockSpec((BR, D), lambda i: (i, 0)),
        ],
        out_shape=[
            jax.ShapeDtypeStruct((B, D), jnp.float32),
            jax.ShapeDtypeStruct((B, D), jnp.float32),
        ],
    )(anchor, modality)

    RB = 256
    loss = pl.pallas_call(
        _loss_kernel,
        grid=(B // RB,),
        in_specs=[
            pl.BlockSpec((RB, D), lambda i: (i, 0)),
            pl.BlockSpec((RB, D), lambda i: (i, 0)),
            pl.BlockSpec((B, D), lambda i: (0, 0)),
            pl.BlockSpec((B, D), lambda i: (0, 0)),
        ],
        out_specs=pl.BlockSpec((1, 1), lambda i: (0, 0)),
        out_shape=jax.ShapeDtypeStruct((1, 1), jnp.float32),
    )(an, mn, an, mn)
    return loss[0, 0]


# speedup vs baseline: 1.0179x; 1.0179x over previous
import jax
import jax.numpy as jnp
from jax.experimental import pallas as pl

_K = 15
_INV_TEMP = 10.0
_EPS = 1e-12


def _pool_norm_kernel(a_ref, m_ref, an_ref, mn_ref):
    for src, dst in ((a_ref, an_ref), (m_ref, mn_ref)):
        x = jnp.mean(src[...], axis=1)
        nrm = jnp.sqrt(jnp.sum(x * x, axis=1, keepdims=True))
        dst[...] = x / jnp.maximum(nrm, _EPS)


def _loss_kernel(ar_ref, mr_ref, afull_ref, mfull_ref, out_ref):
    i = pl.program_id(0)
    rb, b = ar_ref.shape[0], afull_ref.shape[0]
    dn = (((1,), (1,)), ((), ()))
    sim_a = jax.lax.dot_general(ar_ref[...], afull_ref[...], dn,
                                preferred_element_type=jnp.float32)
    row_ids = i * rb + jax.lax.broadcasted_iota(jnp.int32, (rb, b), 0)
    col_ids = jax.lax.broadcasted_iota(jnp.int32, (rb, b), 1)
    is_diag = row_ids == col_ids
    neg_inf = jnp.float32(-jnp.inf)
    sim_a = jnp.where(is_diag, neg_inf, sim_a)

    work = sim_a
    for _ in range(_K - 1):
        mx = jnp.max(work, axis=1, keepdims=True)
        work = jnp.where(work >= mx, neg_inf, work)
    thr = jnp.max(work, axis=1, keepdims=True)
    pos = sim_a >= thr

    sim_m = jax.lax.dot_general(mr_ref[...], mfull_ref[...], dn,
                                preferred_element_type=jnp.float32) * _INV_TEMP
    smax = jnp.max(sim_m, axis=1, keepdims=True)
    e = jnp.exp(sim_m - smax)
    e = jnp.where(is_diag, 0.0, e)
    pos_sum = jnp.sum(jnp.where(pos, e, 0.0), axis=1) + 1e-8
    all_sum = jnp.sum(e, axis=1) + 1e-8
    contrib = jnp.sum(jnp.log(pos_sum) - jnp.log(all_sum))

    @pl.when(i == 0)
    def _():
        out_ref[...] = jnp.zeros_like(out_ref)

    out_ref[...] += jnp.reshape(-contrib / b, (1, 1))


def kernel(anchor, modality):
    B, S, D = anchor.shape
    BR = 32
    an, mn = pl.pallas_call(
        _pool_norm_kernel,
        grid=(B // BR,),
        in_specs=[
            pl.BlockSpec((BR, S, D), lambda i: (i, 0, 0)),
            pl.BlockSpec((BR, S, D), lambda i: (i, 0, 0)),
        ],
        out_specs=[
            pl.BlockSpec((BR, D), lambda i: (i, 0)),
            pl.BlockSpec((BR, D), lambda i: (i, 0)),
        ],
        out_shape=[
            jax.ShapeDtypeStruct((B, D), jnp.float32),
            jax.ShapeDtypeStruct((B, D), jnp.float32),
        ],
    )(anchor, modality)

    RB = 256
    loss = pl.pallas_call(
        _loss_kernel,
        grid=(B // RB,),
        in_specs=[
            pl.BlockSpec((RB, D), lambda i: (i, 0)),
            pl.BlockSpec((RB, D), lambda i: (i, 0)),
            pl.BlockSpec((B, D), lambda i: (0, 0)),
            pl.BlockSpec((B, D), lambda i: (0, 0)),
        ],
        out_specs=pl.BlockSpec((1, 1), lambda i: (0, 0)),
        out_shape=jax.ShapeDtypeStruct((1, 1), jnp.float32),
    )(an, mn, an, mn)
    return loss[0, 0]
